# Initial kernel scaffold; baseline (speedup 1.0000x reference)
import functools

import jax
import jax.numpy as jnp
from jax import lax
from jax.experimental import pallas as pl
from jax.experimental.pallas import tpu as pltpu
from jax.experimental.pallas import tpu_sc as plsc

_N = 32768
_D = 768
_L = 6
_LANES = 16
_NC = 2
_NS = 16
_NW = _NC * _NS
_ROWS_PER_W = _N // _NW
_CHUNK = 64
_NCHUNK = _ROWS_PER_W // _CHUNK
_ACC = _L * _D
_CHUNK_W = _CHUNK * _D


def _sc_body(sm, lb, out, labels_v, buf0, buf1, acc, sem0, sem1):
    wid = lax.axis_index("s") * _NC + lax.axis_index("c")
    row0 = wid * _ROWS_PER_W
    iota = lax.iota(jnp.int32, _LANES)

    pltpu.sync_copy(lb.at[pl.ds(row0, _ROWS_PER_W)], labels_v)

    def _zero(i, _):
        acc[pl.ds(i * _LANES, _LANES)] = jnp.zeros((_LANES,), jnp.float32)
        return 0

    lax.fori_loop(0, _ACC // _LANES, _zero, 0)

    bufs = (buf0, buf1)
    sems = (sem0, sem1)
    base = row0 * _D

    def _start(c):
        return pltpu.async_copy(
            sm.at[pl.ds(base + c * _CHUNK_W, _CHUNK_W)], bufs[c % 2], sems[c % 2]
        )

    cps = [None, None]
    cps[0] = _start(0)
    for c in range(_NCHUNK):
        b = c % 2
        if c + 1 < _NCHUNK:
            cps[(c + 1) % 2] = _start(c + 1)
        cps[b].wait()
        buf = bufs[b]

        def _row(r, _, c_=c, buf_=buf):
            lbl = labels_v[c_ * _CHUNK + r]
            bvec = jnp.full((_LANES,), lbl * _D, jnp.int32) + iota
            off = r * _D
            for j in range(_D // _LANES):
                x = buf_[pl.ds(off + j * _LANES, _LANES)]
                plsc.addupdate_scatter(acc, [bvec + (j * _LANES)], x)
            return 0

        lax.fori_loop(0, _CHUNK, _row, 0)

    pltpu.sync_copy(acc, out.at[wid])


@functools.cache
def _sc_segment_sums():
    mesh = plsc.VectorSubcoreMesh(
        core_axis_name="c", subcore_axis_name="s", num_cores=_NC, num_subcores=_NS
    )
    return pl.kernel(
        _sc_body,
        out_type=jax.ShapeDtypeStruct((_NW, _ACC), jnp.float32),
        mesh=mesh,
        scratch_types=[
            pltpu.VMEM((_ROWS_PER_W,), jnp.int32),
            pltpu.VMEM((_CHUNK_W,), jnp.float32),
            pltpu.VMEM((_CHUNK_W,), jnp.float32),
            pltpu.VMEM((_ACC,), jnp.float32),
            pltpu.SemaphoreType.DMA,
            pltpu.SemaphoreType.DMA,
        ],
    )


def _finish_body(parts_ref, labels_ref, out_ref):
    parts = parts_ref[...]
    sums = functools.reduce(
        lambda a, i: a + parts[_L * i : _L * (i + 1)], range(1, _NW), parts[0:_L]
    )
    lab = labels_ref[...]
    cs = [jnp.sum(jnp.where(lab == l, 1.0, 0.0)) for l in range(_L)]
    pres = [jnp.where(c > 0.0, 1.0, 0.0) for c in cs]
    npres = functools.reduce(lambda a, b: a + b, pres)
    cmat = jnp.concatenate(
        [jnp.full((1, _D), jnp.maximum(c, 1.0)) for c in cs], axis=0
    )
    pmat = jnp.concatenate([jnp.full((1, _D), p) for p in pres], axis=0)
    means = sums / cmat
    mean_over = jnp.sum(means * pmat, axis=0, keepdims=True) / npres
    col_var = (
        jnp.sum(((means - mean_over) ** 2) * pmat, axis=0, keepdims=True) / npres
    )
    mv = jnp.mean(col_var)
    mx = jnp.max(col_var)
    mn = jnp.min(col_var)
    loss = jnp.where(npres >= 2.0, -mv, 0.0)
    out_ref[...] = jnp.concatenate(
        [jnp.full((1, 128), v, jnp.float32) for v in (loss, mv, mx, mn)]
        + [jnp.zeros((4, 128), jnp.float32)],
        axis=0,
    )


@functools.cache
def _finish():
    return pl.pallas_call(
        _finish_body,
        out_shape=jax.ShapeDtypeStruct((8, 128), jnp.float32),
    )


def kernel(soft_mask, bloom_labels):
    sm_flat = soft_mask.reshape(-1)
    parts = _sc_segment_sums()(sm_flat, bloom_labels)
    o = _finish()(parts.reshape(_NW * _L, _D), bloom_labels.reshape(256, 128))
    return (o[0, 0], o[1, 0], o[2, 0], o[3, 0])


# speedup vs baseline: 1.0670x; 1.0670x over previous
import functools

import jax
import jax.numpy as jnp
from jax import lax
from jax.experimental import pallas as pl
from jax.experimental.pallas import tpu as pltpu
from jax.experimental.pallas import tpu_sc as plsc

_N = 32768
_D = 768
_L = 6
_LANES = 16
_NC = 2
_NS = 16
_NW = _NC * _NS
_ROWS_PER_W = _N // _NW
_CHUNK = 64
_NCHUNK = _ROWS_PER_W // _CHUNK
_ACC = _L * _D
_CHUNK_W = _CHUNK * _D


def _sc_body(sm, lb, out, labels_v, buf0, buf1, acc, sem0, sem1):
    wid = lax.axis_index("s") * _NC + lax.axis_index("c")
    row0 = wid * _ROWS_PER_W
    iota = lax.iota(jnp.int32, _LANES)

    pltpu.sync_copy(lb.at[pl.ds(row0, _ROWS_PER_W)], labels_v.at[pl.ds(0, _ROWS_PER_W)])

    def _zero(i, _):
        acc[pl.ds(i * _LANES, _LANES)] = jnp.zeros((_LANES,), jnp.float32)
        return 0

    lax.fori_loop(0, _ACC // _LANES, _zero, 0)

    bufs = (buf0, buf1)
    sems = (sem0, sem1)
    base = row0 * _D

    def _start(c):
        return pltpu.async_copy(
            sm.at[pl.ds(base + c * _CHUNK_W, _CHUNK_W)], bufs[c % 2], sems[c % 2]
        )

    cps = [None, None]
    cps[0] = _start(0)
    for c in range(_NCHUNK):
        b = c % 2
        if c + 1 < _NCHUNK:
            cps[(c + 1) % 2] = _start(c + 1)
        cps[b].wait()
        buf = bufs[b]

        def _row(r, _, c_=c, buf_=buf):
            lbl = labels_v[pl.ds(c_ * _CHUNK + r, _LANES)][0]
            bvec = jnp.full((_LANES,), lbl * _D, jnp.int32) + iota
            off = r * _D
            for j in range(_D // _LANES):
                x = buf_[pl.ds(off + j * _LANES, _LANES)]
                plsc.addupdate_scatter(acc, [bvec + (j * _LANES)], x)
            return 0

        lax.fori_loop(0, _CHUNK, _row, 0)

    pltpu.sync_copy(acc, out.at[wid])


@functools.cache
def _sc_segment_sums():
    mesh = plsc.VectorSubcoreMesh(
        core_axis_name="c", subcore_axis_name="s", num_cores=_NC, num_subcores=_NS
    )
    return pl.kernel(
        _sc_body,
        out_type=jax.ShapeDtypeStruct((_NW, _ACC), jnp.float32),
        mesh=mesh,
        scratch_types=[
            pltpu.VMEM((_ROWS_PER_W + _LANES,), jnp.int32),
            pltpu.VMEM((_CHUNK_W,), jnp.float32),
            pltpu.VMEM((_CHUNK_W,), jnp.float32),
            pltpu.VMEM((_ACC,), jnp.float32),
            pltpu.SemaphoreType.DMA,
            pltpu.SemaphoreType.DMA,
        ],
        compiler_params=pltpu.CompilerParams(needs_layout_passes=False),
    )


def _finish_body(parts_ref, labels_ref, out_ref):
    parts = parts_ref[...]
    sums = functools.reduce(
        lambda a, i: a + parts[_L * i : _L * (i + 1)], range(1, _NW), parts[0:_L]
    )
    lab = labels_ref[...]
    cs = [jnp.sum(jnp.where(lab == l, 1.0, 0.0)) for l in range(_L)]
    pres = [jnp.where(c > 0.0, 1.0, 0.0) for c in cs]
    npres = functools.reduce(lambda a, b: a + b, pres)
    cmat = jnp.concatenate(
        [jnp.full((1, _D), jnp.maximum(c, 1.0)) for c in cs], axis=0
    )
    pmat = jnp.concatenate([jnp.full((1, _D), p) for p in pres], axis=0)
    means = sums / cmat
    mean_over = jnp.sum(means * pmat, axis=0, keepdims=True) / npres
    col_var = (
        jnp.sum(((means - mean_over) ** 2) * pmat, axis=0, keepdims=True) / npres
    )
    mv = jnp.mean(col_var)
    mx = jnp.max(col_var)
    mn = jnp.min(col_var)
    loss = jnp.where(npres >= 2.0, -mv, 0.0)
    out_ref[...] = jnp.concatenate(
        [jnp.full((1, 128), v, jnp.float32) for v in (loss, mv, mx, mn)]
        + [jnp.zeros((4, 128), jnp.float32)],
        axis=0,
    )


@functools.cache
def _finish():
    return pl.pallas_call(
        _finish_body,
        out_shape=jax.ShapeDtypeStruct((8, 128), jnp.float32),
    )


def kernel(soft_mask, bloom_labels):
    sm_flat = soft_mask.reshape(-1)
    parts = _sc_segment_sums()(sm_flat, bloom_labels)
    o = _finish()(parts.reshape(_NW * _L, _D), bloom_labels.reshape(256, 128))
    return (o[0, 0], o[1, 0], o[2, 0], o[3, 0])
